# baseline (speedup 1.0000x reference)
import functools

import jax
import jax.numpy as jnp
from jax import lax
from jax.experimental import pallas as pl
from jax.experimental.pallas import tpu as pltpu
from jax.experimental.pallas import tpu_sc as plsc

_LEVEL_SIZES = (16, 48, 192, 768, 3072, 8192, 10240, 10240)
_N = sum(_LEVEL_SIZES)
_D = 128
_BS = 16
_NC = 2
_NS = 16
_NW = _NC * _NS

_OFFS = [0]
for _s in _LEVEL_SIZES:
    _OFFS.append(_OFFS[-1] + _s)

_GCH = 128
_ROWS_PER_TILE = _N // _NW
_GN = _ROWS_PER_TILE // _GCH
_PCH = 256
_MMB = 512


def _mesh():
    return plsc.VectorSubcoreMesh(core_axis_name="c", subcore_axis_name="s")


@functools.partial(
    pl.kernel,
    out_type=jax.ShapeDtypeStruct((_N, _D), jnp.float32),
    mesh=_mesh(),
    scratch_types=[
        pltpu.VMEM((_GN, _GCH), jnp.int32),
        pltpu.VMEM((2, _GCH, _D), jnp.float32),
        pltpu.SemaphoreType.DMA,
        pltpu.SemaphoreType.DMA,
    ],
)
def _gather(table_hbm, tok_hbm, out_hbm, idx_v, bufs, sem0, sem1):
    wid = lax.axis_index("s") * _NC + lax.axis_index("c")
    pltpu.sync_copy(tok_hbm.at[pl.ds(wid * _GN, _GN)], idx_v)
    sems = (sem0, sem1)
    cps = [None, None]
    cps[0] = pltpu.async_copy(table_hbm.at[idx_v.at[0]], bufs.at[0], sem0)
    for c in range(_GN):
        cur = c % 2
        if c + 1 < _GN:
            cps[1 - cur] = pltpu.async_copy(
                table_hbm.at[idx_v.at[c + 1]], bufs.at[1 - cur], sems[1 - cur])
        cps[cur].wait()
        pltpu.sync_copy(
            bufs.at[cur],
            out_hbm.at[pl.ds(wid * _ROWS_PER_TILE + c * _GCH, _GCH)])


def _masked_slot_max(ids, x_, acc):
    parts = []
    for s2 in range(_BS):
        parts.append(jnp.max(jnp.where(ids == s2, x_, -1e30), axis=0,
                             keepdims=True))
    return jnp.maximum(acc, jnp.concatenate(parts, axis=0))


def _matmul(emb, w, b2, bi2):
    leaf_blk = _NINT // _MMB

    def body(x_ref, w_ref, b_ref, ids_ref, o_ref, m_ref):
        pid = pl.program_id(0)
        h_ = lax.dot_general(
            x_ref[...], w_ref[...], (((1,), (1,)), ((), ())),
            preferred_element_type=jnp.float32) + b_ref[...]
        o_ref[...] = h_

        @pl.when(pid == 0)
        def _():
            m_ref[...] = jnp.zeros((_BS, _D), jnp.float32)

        @pl.when(pid >= leaf_blk)
        def _():
            m_ref[...] = _masked_slot_max(ids_ref[...], h_, m_ref[...])

    return pl.pallas_call(
        body,
        grid=(_N // _MMB,),
        in_specs=[
            pl.BlockSpec((_MMB, _D), lambda i: (i, 0)),
            pl.BlockSpec((_D, _D), lambda i: (0, 0)),
            pl.BlockSpec((1, _D), lambda i: (0, 0)),
            pl.BlockSpec((_MMB, 1), lambda i: (i, 0)),
        ],
        out_specs=[
            pl.BlockSpec((_MMB, _D), lambda i: (i, 0)),
            pl.BlockSpec((_BS, _D), lambda i: (0, 0)),
        ],
        out_shape=[
            jax.ShapeDtypeStruct((_N, _D), jnp.float32),
            jax.ShapeDtypeStruct((_BS, _D), jnp.float32),
        ],
    )(emb, w, b2, bi2)


_PMAXROWS = 5120
_NINT = 22528


@functools.partial(
    pl.kernel,
    out_type=jax.ShapeDtypeStruct((_NINT, _D), jnp.float32),
    mesh=_mesh(),
    scratch_types=[
        pltpu.VMEM((5, _GCH), jnp.int32),
        pltpu.VMEM((640, _D), jnp.float32),
        pltpu.VMEM_SHARED((_PMAXROWS + 8, _D), jnp.float32),
        pltpu.SemaphoreType.DMA,
    ],
)
def _tree(h_hbm, pp_hbm, hout_hbm, idx2d, buf, partial, sem):
    cid = lax.axis_index("c")
    tid = lax.axis_index("s")
    on0 = cid == 0

    nlev = len(_LEVEL_SIZES)
    for l in range(nlev - 1, 0, -1):
        s, n = _OFFS[l], _LEVEL_SIZES[l]
        ps, pn = _OFFS[l - 1], _LEVEL_SIZES[l - 1]
        src = h_hbm if l == nlev - 1 else hout_hbm
        if pn > _PMAXROWS:
            passes = [(0, pn // 2), (pn // 2, pn // 2)]
        else:
            passes = [(0, pn)]
        A = min(_NS, n // _GCH) if n >= _GCH else 1
        span = n // A
        full, tail = span // _GCH, span % _GCH

        for lo, np_ in passes:
            clamp = len(passes) > 1
            ap = max(a for a in range(1, _NS + 1)
                     if np_ % a == 0 and (np_ // a) % 8 == 0)
            pspan = np_ // ap

            def prow(ref, base):
                off = pl.multiple_of(base + tid * pspan, 8)
                return ref.at[pl.ds(off, pspan)]

            def srow():
                off = pl.multiple_of(tid * pspan, 8)
                return partial.at[pl.ds(off, pspan)]

            bslice = buf.at[pl.ds(0, pspan)]

            pon = jnp.logical_and(on0, tid < ap) if ap < _NS else on0

            def p1(b=ps + lo, prow=prow, srow=srow, bslice=bslice):
                pltpu.sync_copy(prow(h_hbm, b), bslice)
                pltpu.sync_copy(bslice, srow())

            pl.when(pon)(p1)
            plsc.subcore_barrier()

            def p2(s=s, src=src, lo=lo, np_=np_, clamp=clamp, span=span,
                   full=full, tail=tail):
                cbase = pl.multiple_of(s + tid * span, 8)
                ppcps = [pltpu.async_copy(
                    pp_hbm.at[pl.ds(cbase + j * _GCH, _GCH)], idx2d.at[j],
                    sem) for j in range(full)]
                pltpu.sync_copy(src.at[pl.ds(cbase, span)],
                                buf.at[pl.ds(0, span)])
                for cp in ppcps:
                    cp.wait()
                if clamp:
                    for j in range(full):
                        for q in range(_GCH // 16):
                            v = idx2d[j, pl.ds(q * 16, 16)]
                            ok = jnp.logical_and(v >= lo, v < lo + np_)
                            idx2d[j, pl.ds(q * 16, 16)] = jnp.where(
                                ok, v - lo, jnp.int32(np_))
                for j in range(full):
                    pltpu.sync_copy(buf.at[pl.ds(j * _GCH, _GCH)],
                                    partial.at[idx2d.at[j]], add=True)
                if tail:
                    def scoped(idxs):
                        pltpu.sync_copy(
                            pp_hbm.at[pl.ds(cbase + full * _GCH, tail)], idxs)
                        pltpu.sync_copy(buf.at[pl.ds(full * _GCH, tail)],
                                        partial.at[idxs], add=True)
                    pl.run_scoped(scoped, pltpu.VMEM((tail,), jnp.int32))

            pl.when(jnp.logical_and(on0, tid < A) if A < _NS else on0)(p2)
            plsc.subcore_barrier()

            def p3(b=ps + lo, prow=prow, srow=srow, bslice=bslice):
                pltpu.sync_copy(srow(), bslice)
                pltpu.sync_copy(bslice, prow(hout_hbm, b))

            pl.when(pon)(p3)
            plsc.subcore_barrier()


def _segmax(bi2, x, init, base_blk, nrows):
    def body(ids_ref, x_ref, init_ref, o_ref):
        pid = pl.program_id(0)

        @pl.when(pid == 0)
        def _():
            o_ref[...] = init_ref[...]

        o_ref[...] = _masked_slot_max(ids_ref[...], x_ref[...], o_ref[...])

    return pl.pallas_call(
        body,
        grid=(nrows // _MMB,),
        in_specs=[
            pl.BlockSpec((_MMB, 1), lambda i, b=base_blk: (b + i, 0)),
            pl.BlockSpec((_MMB, _D), lambda i, b=base_blk: (b + i, 0)),
            pl.BlockSpec((_BS, _D), lambda i: (0, 0)),
        ],
        out_specs=pl.BlockSpec((_BS, _D), lambda i: (0, 0)),
        out_shape=jax.ShapeDtypeStruct((_BS, _D), jnp.float32),
    )(bi2, x, init)


def kernel(tokens, level_offsets, parent_pos, batch_index, bs, table, W, b):
    del level_offsets, bs
    tok2 = tokens.astype(jnp.int32).reshape(_N // _GCH, _GCH)
    pp32 = parent_pos.astype(jnp.int32)
    bi2 = batch_index.astype(jnp.int32).reshape(_N, 1)
    emb = _gather(table, tok2)
    h, part = _matmul(emb, W, b.reshape(1, _D), bi2)
    hout = _tree(h, pp32)
    return _segmax(bi2, hout, part, 0, _NINT)


# speedup vs baseline: 1.1608x; 1.1608x over previous
import functools

import jax
import jax.numpy as jnp
from jax import lax
from jax.experimental import pallas as pl
from jax.experimental.pallas import tpu as pltpu
from jax.experimental.pallas import tpu_sc as plsc

_LEVEL_SIZES = (16, 48, 192, 768, 3072, 8192, 10240, 10240)
_N = sum(_LEVEL_SIZES)
_D = 128
_BS = 16
_NC = 2
_NS = 16
_NW = _NC * _NS

_OFFS = [0]
for _s in _LEVEL_SIZES:
    _OFFS.append(_OFFS[-1] + _s)

_GCH = 128
_ROWS_PER_TILE = _N // _NW
_GN = _ROWS_PER_TILE // _GCH
_MMB = 512
_PMAXROWS = 5120
_NINT = _OFFS[-2]


def _mesh():
    return plsc.VectorSubcoreMesh(core_axis_name="c", subcore_axis_name="s")


@functools.partial(
    pl.kernel,
    out_type=jax.ShapeDtypeStruct((_N, _D), jnp.float32),
    mesh=_mesh(),
    scratch_types=[
        pltpu.VMEM((_GN, _GCH), jnp.int32),
        pltpu.VMEM((2, _GCH, _D), jnp.float32),
        pltpu.SemaphoreType.DMA,
        pltpu.SemaphoreType.DMA,
    ],
)
def _gather(table_hbm, tok_hbm, out_hbm, idx_v, bufs, sem0, sem1):
    wid = lax.axis_index("s") * _NC + lax.axis_index("c")
    pltpu.sync_copy(tok_hbm.at[pl.ds(wid * _GN, _GN)], idx_v)
    sems = (sem0, sem1)
    cps = [None, None]
    cps[0] = pltpu.async_copy(table_hbm.at[idx_v.at[0]], bufs.at[0], sem0)
    for c in range(_GN):
        cur = c % 2
        if c + 1 < _GN:
            cps[1 - cur] = pltpu.async_copy(
                table_hbm.at[idx_v.at[c + 1]], bufs.at[1 - cur], sems[1 - cur])
        cps[cur].wait()
        pltpu.sync_copy(
            bufs.at[cur],
            out_hbm.at[pl.ds(wid * _ROWS_PER_TILE + c * _GCH, _GCH)])


def _matmul(emb, w, b2):
    def body(x_ref, w_ref, b_ref, o_ref):
        o_ref[...] = lax.dot_general(
            x_ref[...], w_ref[...], (((1,), (1,)), ((), ())),
            preferred_element_type=jnp.float32) + b_ref[...]

    return pl.pallas_call(
        body,
        grid=(_N // _MMB,),
        in_specs=[
            pl.BlockSpec((_MMB, _D), lambda i: (i, 0)),
            pl.BlockSpec((_D, _D), lambda i: (0, 0)),
            pl.BlockSpec((1, _D), lambda i: (0, 0)),
        ],
        out_specs=pl.BlockSpec((_MMB, _D), lambda i: (i, 0)),
        out_shape=jax.ShapeDtypeStruct((_N, _D), jnp.float32),
    )(emb, w, b2)


def _make_tree_stage(levels, split, csrc_base):
    first = levels[0]
    out_base = _OFFS[levels[-1] - 1]
    out_rows = _OFFS[first] - out_base

    @functools.partial(
        pl.kernel,
        out_type=jax.ShapeDtypeStruct((out_rows, _D), jnp.float32),
        mesh=_mesh(),
        scratch_types=[
            pltpu.VMEM((5, _GCH), jnp.int32),
            pltpu.VMEM((640, _D), jnp.float32),
            pltpu.VMEM_SHARED((_PMAXROWS + 8, _D), jnp.float32),
            pltpu.SemaphoreType.DMA,
        ],
    )
    def stage(h_hbm, csrc_hbm, pp_hbm, out_hbm, idx2d, buf, partial, sem):
        cid = lax.axis_index("c")
        tid = lax.axis_index("s")
        on0 = cid == 0

        for li, l in enumerate(levels):
            s, n = _OFFS[l], _LEVEL_SIZES[l]
            ps, pn = _OFFS[l - 1], _LEVEL_SIZES[l - 1]
            if li == 0:
                src, sbase = csrc_hbm, csrc_base
            else:
                src, sbase = out_hbm, out_base
            if split:
                half = pn // _NC
                passes = [(cid * half, half)]
                np_ = half
            else:
                assert pn <= _PMAXROWS
                passes = [(0, pn)]
                np_ = pn
            A = min(_NS, n // _GCH) if n >= _GCH else 1
            span = n // A
            full, tail = span // _GCH, span % _GCH

            for lo, np_ in passes:
                clamp = np_ < pn
                ap = max(a for a in range(1, _NS + 1)
                         if np_ % a == 0 and (np_ // a) % 8 == 0)
                pspan = np_ // ap

                def prow(ref, base):
                    off = pl.multiple_of(base + tid * pspan, 8)
                    return ref.at[pl.ds(off, pspan)]

                def srow():
                    off = pl.multiple_of(tid * pspan, 8)
                    return partial.at[pl.ds(off, pspan)]

                bslice = buf.at[pl.ds(0, pspan)]
                if split:
                    pon = tid < ap if ap < _NS else None
                    won = tid < A if A < _NS else None
                else:
                    pon = jnp.logical_and(on0, tid < ap) if ap < _NS else on0
                    won = jnp.logical_and(on0, tid < A) if A < _NS else on0

                def _guard(pred, thunk):
                    if pred is None:
                        thunk()
                    else:
                        pl.when(pred)(thunk)

                def p1(b=ps, lo=lo, prow=prow, srow=srow, bslice=bslice):
                    pltpu.sync_copy(prow(h_hbm, b + lo), bslice)
                    pltpu.sync_copy(bslice, srow())

                _guard(pon, p1)
                plsc.subcore_barrier()

                def p2(s=s, src=src, sbase=sbase, lo=lo, np_=np_,
                       clamp=clamp, span=span, full=full, tail=tail):
                    gbase = pl.multiple_of(s + tid * span, 8)
                    lbase = pl.multiple_of(s - sbase + tid * span, 8)
                    ppcps = [pltpu.async_copy(
                        pp_hbm.at[pl.ds(gbase + j * _GCH, _GCH)], idx2d.at[j],
                        sem) for j in range(full)]
                    pltpu.sync_copy(src.at[pl.ds(lbase, span)],
                                    buf.at[pl.ds(0, span)])
                    for cp in ppcps:
                        cp.wait()
                    if clamp:
                        for j in range(full):
                            for q in range(_GCH // 16):
                                v = idx2d[j, pl.ds(q * 16, 16)]
                                ok = jnp.logical_and(v >= lo, v < lo + np_)
                                idx2d[j, pl.ds(q * 16, 16)] = jnp.where(
                                    ok, v - lo, jnp.int32(np_))
                    for j in range(full):
                        pltpu.sync_copy(buf.at[pl.ds(j * _GCH, _GCH)],
                                        partial.at[idx2d.at[j]], add=True)
                    if tail:
                        def scoped(idxs):
                            pltpu.sync_copy(
                                pp_hbm.at[pl.ds(gbase + full * _GCH, tail)],
                                idxs)
                            pltpu.sync_copy(buf.at[pl.ds(full * _GCH, tail)],
                                            partial.at[idxs], add=True)
                        pl.run_scoped(scoped, pltpu.VMEM((tail,), jnp.int32))

                _guard(won, p2)
                plsc.subcore_barrier()

                def p3(b=ps - out_base, lo=lo, prow=prow, srow=srow,
                       bslice=bslice):
                    pltpu.sync_copy(srow(), bslice)
                    pltpu.sync_copy(bslice, prow(out_hbm, b + lo))

                _guard(pon, p3)
                plsc.subcore_barrier()

    return stage


_T1 = _make_tree_stage([7], split=True, csrc_base=0)
_T2 = _make_tree_stage([6], split=True, csrc_base=_OFFS[6])
_T3 = _make_tree_stage([5, 4, 3, 2, 1], split=False,
                       csrc_base=_OFFS[5])


def _segmax(bi2, x, init, ids_blk, nrows, x_blk=0):
    def body(ids_ref, x_ref, init_ref, o_ref):
        pid = pl.program_id(0)

        @pl.when(pid == 0)
        def _():
            o_ref[...] = init_ref[...]

        x_ = x_ref[...]
        ids = ids_ref[...]
        parts = []
        for s2 in range(_BS):
            parts.append(jnp.max(jnp.where(ids == s2, x_, -1e30), axis=0,
                                 keepdims=True))
        o_ref[...] = jnp.maximum(o_ref[...], jnp.concatenate(parts, axis=0))

    return pl.pallas_call(
        body,
        grid=(nrows // _MMB,),
        in_specs=[
            pl.BlockSpec((_MMB, 1), lambda i, b=ids_blk: (b + i, 0)),
            pl.BlockSpec((_MMB, _D), lambda i, b=x_blk: (b + i, 0)),
            pl.BlockSpec((_BS, _D), lambda i: (0, 0)),
        ],
        out_specs=pl.BlockSpec((_BS, _D), lambda i: (0, 0)),
        out_shape=jax.ShapeDtypeStruct((_BS, _D), jnp.float32),
    )(bi2, x, init)


def kernel(tokens, level_offsets, parent_pos, batch_index, bs, table, W, b):
    del level_offsets, bs
    tok2 = tokens.astype(jnp.int32).reshape(_N // _GCH, _GCH)
    pp32 = parent_pos.astype(jnp.int32)
    bi2 = batch_index.astype(jnp.int32).reshape(_N, 1)
    emb = _gather(table, tok2)
    h = _matmul(emb, W, b.reshape(1, _D))
    lvl6 = _T1(h, h, pp32)
    part = _segmax(bi2, h, jnp.zeros((_BS, _D), jnp.float32), _NINT // _MMB,
                   _N - _NINT, x_blk=_NINT // _MMB)
    lvl5 = _T2(h, lvl6, pp32)
    part = _segmax(bi2, lvl6, part, _OFFS[6] // _MMB, _LEVEL_SIZES[6])
    rest = _T3(h, lvl5, pp32)
    part = _segmax(bi2, lvl5, part, _OFFS[5] // _MMB, _LEVEL_SIZES[5])
    return _segmax(bi2, rest, part, 0, _OFFS[5])


# speedup vs baseline: 1.3132x; 1.1313x over previous
import functools

import jax
import jax.numpy as jnp
from jax import lax
from jax.experimental import pallas as pl
from jax.experimental.pallas import tpu as pltpu
from jax.experimental.pallas import tpu_sc as plsc

_LEVEL_SIZES = (16, 48, 192, 768, 3072, 8192, 10240, 10240)
_N = sum(_LEVEL_SIZES)
_D = 128
_BS = 16
_NC = 2
_NS = 16
_NW = _NC * _NS

_OFFS = [0]
for _s in _LEVEL_SIZES:
    _OFFS.append(_OFFS[-1] + _s)

_GCH = 128
_ROWS_PER_TILE = _N // _NW
_GN = _ROWS_PER_TILE // _GCH
_MMB = 512
_PMAXROWS = 5120
_NINT = _OFFS[-2]


def _mesh():
    return plsc.VectorSubcoreMesh(core_axis_name="c", subcore_axis_name="s")


@functools.partial(
    pl.kernel,
    out_type=jax.ShapeDtypeStruct((_N, _D), jnp.float32),
    mesh=_mesh(),
    scratch_types=[
        pltpu.VMEM((_GN, _GCH), jnp.int32),
        pltpu.VMEM((2, _GCH, _D), jnp.float32),
        pltpu.SemaphoreType.DMA,
        pltpu.SemaphoreType.DMA,
    ],
)
def _gather(table_hbm, tok_hbm, out_hbm, idx_v, bufs, sem0, sem1):
    wid = lax.axis_index("s") * _NC + lax.axis_index("c")
    pltpu.sync_copy(tok_hbm.at[pl.ds(wid * _GN, _GN)], idx_v)
    sems = (sem0, sem1)
    cps = [None, None]
    cps[0] = pltpu.async_copy(table_hbm.at[idx_v.at[0]], bufs.at[0], sem0)
    for c in range(_GN):
        cur = c % 2
        if c + 1 < _GN:
            cps[1 - cur] = pltpu.async_copy(
                table_hbm.at[idx_v.at[c + 1]], bufs.at[1 - cur], sems[1 - cur])
        cps[cur].wait()
        pltpu.sync_copy(
            bufs.at[cur],
            out_hbm.at[pl.ds(wid * _ROWS_PER_TILE + c * _GCH, _GCH)])


_MMBLK = 2048


def _matmul(emb, w, b2):
    def body(x_ref, w_ref, b_ref, o_ref):
        o_ref[...] = lax.dot_general(
            x_ref[...], w_ref[...], (((1,), (1,)), ((), ())),
            preferred_element_type=jnp.float32) + b_ref[...]

    return pl.pallas_call(
        body,
        grid=(_N // _MMBLK,),
        in_specs=[
            pl.BlockSpec((_MMBLK, _D), lambda i: (i, 0)),
            pl.BlockSpec((_D, _D), lambda i: (0, 0)),
            pl.BlockSpec((1, _D), lambda i: (0, 0)),
        ],
        out_specs=pl.BlockSpec((_MMBLK, _D), lambda i: (i, 0)),
        out_shape=jax.ShapeDtypeStruct((_N, _D), jnp.float32),
    )(emb, w, b2)


def _make_tree_stage(levels, split, csrc_base):
    first = levels[0]
    out_base = _OFFS[levels[-1] - 1]
    out_rows = _OFFS[first] - out_base

    @functools.partial(
        pl.kernel,
        out_type=jax.ShapeDtypeStruct((out_rows, _D), jnp.float32),
        mesh=_mesh(),
        scratch_types=[
            pltpu.VMEM((5, _GCH), jnp.int32),
            pltpu.VMEM((640, _D), jnp.float32),
            pltpu.VMEM_SHARED((_PMAXROWS + 8, _D), jnp.float32),
            pltpu.SemaphoreType.DMA,
        ],
    )
    def stage(h_hbm, csrc_hbm, pp_hbm, out_hbm, idx2d, buf, partial, sem):
        cid = lax.axis_index("c")
        tid = lax.axis_index("s")
        on0 = cid == 0

        for li, l in enumerate(levels):
            s, n = _OFFS[l], _LEVEL_SIZES[l]
            ps, pn = _OFFS[l - 1], _LEVEL_SIZES[l - 1]
            if li == 0:
                src, sbase = csrc_hbm, csrc_base
            else:
                src, sbase = out_hbm, out_base
            if split:
                half = pn // _NC
                passes = [(cid * half, half)]
                np_ = half
            else:
                assert pn <= _PMAXROWS
                passes = [(0, pn)]
                np_ = pn
            A = min(_NS, n // _GCH) if n >= _GCH else 1
            span = n // A
            full, tail = span // _GCH, span % _GCH

            for lo, np_ in passes:
                clamp = np_ < pn
                ap = max(a for a in range(1, _NS + 1)
                         if np_ % a == 0 and (np_ // a) % 8 == 0)
                pspan = np_ // ap

                def prow(ref, base):
                    off = pl.multiple_of(base + tid * pspan, 8)
                    return ref.at[pl.ds(off, pspan)]

                def srow():
                    off = pl.multiple_of(tid * pspan, 8)
                    return partial.at[pl.ds(off, pspan)]

                bslice = buf.at[pl.ds(0, pspan)]
                if split:
                    pon = tid < ap if ap < _NS else None
                    won = tid < A if A < _NS else None
                else:
                    pon = jnp.logical_and(on0, tid < ap) if ap < _NS else on0
                    won = jnp.logical_and(on0, tid < A) if A < _NS else on0

                def _guard(pred, thunk):
                    if pred is None:
                        thunk()
                    else:
                        pl.when(pred)(thunk)

                def p1(b=ps, lo=lo, prow=prow, srow=srow, bslice=bslice):
                    pltpu.sync_copy(prow(h_hbm, b + lo), bslice)
                    pltpu.sync_copy(bslice, srow())

                _guard(pon, p1)
                plsc.subcore_barrier()

                def p2(s=s, src=src, sbase=sbase, lo=lo, np_=np_,
                       clamp=clamp, span=span, full=full, tail=tail):
                    gbase = pl.multiple_of(s + tid * span, 8)
                    lbase = pl.multiple_of(s - sbase + tid * span, 8)
                    ppcps = [pltpu.async_copy(
                        pp_hbm.at[pl.ds(gbase + j * _GCH, _GCH)], idx2d.at[j],
                        sem) for j in range(full)]
                    pltpu.sync_copy(src.at[pl.ds(lbase, span)],
                                    buf.at[pl.ds(0, span)])
                    for cp in ppcps:
                        cp.wait()
                    if clamp:
                        for j in range(full):
                            for q in range(_GCH // 16):
                                v = idx2d[j, pl.ds(q * 16, 16)]
                                ok = jnp.logical_and(v >= lo, v < lo + np_)
                                idx2d[j, pl.ds(q * 16, 16)] = jnp.where(
                                    ok, v - lo, jnp.int32(np_))
                    for j in range(full):
                        pltpu.sync_copy(buf.at[pl.ds(j * _GCH, _GCH)],
                                        partial.at[idx2d.at[j]], add=True)
                    if tail:
                        def scoped(idxs):
                            pltpu.sync_copy(
                                pp_hbm.at[pl.ds(gbase + full * _GCH, tail)],
                                idxs)
                            pltpu.sync_copy(buf.at[pl.ds(full * _GCH, tail)],
                                            partial.at[idxs], add=True)
                        pl.run_scoped(scoped, pltpu.VMEM((tail,), jnp.int32))

                _guard(won, p2)
                plsc.subcore_barrier()

                def p3(b=ps - out_base, lo=lo, prow=prow, srow=srow,
                       bslice=bslice):
                    pltpu.sync_copy(srow(), bslice)
                    pltpu.sync_copy(bslice, prow(out_hbm, b + lo))

                _guard(pon, p3)
                plsc.subcore_barrier()

    return stage


_T1 = _make_tree_stage([7], split=True, csrc_base=0)
_T2 = _make_tree_stage([6], split=True, csrc_base=_OFFS[6])
_T3 = _make_tree_stage([5, 4, 3, 2, 1], split=False,
                       csrc_base=_OFFS[5])


def _segmax(bi2, x, init, ids_blk, nrows, x_blk=0):
    def body(ids_ref, x_ref, init_ref, o_ref):
        pid = pl.program_id(0)

        @pl.when(pid == 0)
        def _():
            o_ref[...] = init_ref[...]

        x_ = x_ref[...]
        ids = ids_ref[...]
        parts = []
        for s2 in range(_BS):
            parts.append(jnp.max(jnp.where(ids == s2, x_, -1e30), axis=0,
                                 keepdims=True))
        o_ref[...] = jnp.maximum(o_ref[...], jnp.concatenate(parts, axis=0))

    return pl.pallas_call(
        body,
        grid=(nrows // _MMB,),
        in_specs=[
            pl.BlockSpec((_MMB, 1), lambda i, b=ids_blk: (b + i, 0)),
            pl.BlockSpec((_MMB, _D), lambda i, b=x_blk: (b + i, 0)),
            pl.BlockSpec((_BS, _D), lambda i: (0, 0)),
        ],
        out_specs=pl.BlockSpec((_BS, _D), lambda i: (0, 0)),
        out_shape=jax.ShapeDtypeStruct((_BS, _D), jnp.float32),
    )(bi2, x, init)


def kernel(tokens, level_offsets, parent_pos, batch_index, bs, table, W, b):
    del level_offsets, bs
    tok2 = tokens.astype(jnp.int32).reshape(_N // _GCH, _GCH)
    pp32 = parent_pos.astype(jnp.int32)
    bi2 = batch_index.astype(jnp.int32).reshape(_N, 1)
    emb = _gather(table, tok2)
    h = _matmul(emb, W, b.reshape(1, _D))
    lvl6 = _T1(h, h, pp32)
    part = _segmax(bi2, h, jnp.zeros((_BS, _D), jnp.float32), _NINT // _MMB,
                   _N - _NINT, x_blk=_NINT // _MMB)
    lvl5 = _T2(h, lvl6, pp32)
    part = _segmax(bi2, lvl6, part, _OFFS[6] // _MMB, _LEVEL_SIZES[6])
    rest = _T3(h, lvl5, pp32)
    part = _segmax(bi2, lvl5, part, _OFFS[5] // _MMB, _LEVEL_SIZES[5])
    return _segmax(bi2, rest, part, 0, _OFFS[5])


# speedup vs baseline: 1.5000x; 1.1422x over previous
import functools

import jax
import jax.numpy as jnp
from jax import lax
from jax.experimental import pallas as pl
from jax.experimental.pallas import tpu as pltpu
from jax.experimental.pallas import tpu_sc as plsc

_LEVEL_SIZES = (16, 48, 192, 768, 3072, 8192, 10240, 10240)
_N = sum(_LEVEL_SIZES)
_D = 128
_BS = 16
_NC = 2
_NS = 16
_NW = _NC * _NS

_OFFS = [0]
for _s in _LEVEL_SIZES:
    _OFFS.append(_OFFS[-1] + _s)

_GCH = 128
_ROWS_PER_TILE = _N // _NW
_GN = _ROWS_PER_TILE // _GCH
_MMB = 512
_PMAXROWS = 5120
_NINT = _OFFS[-2]


def _mesh():
    return plsc.VectorSubcoreMesh(core_axis_name="c", subcore_axis_name="s")


@functools.partial(
    pl.kernel,
    out_type=jax.ShapeDtypeStruct((_N, _D), jnp.float32),
    mesh=_mesh(),
    scratch_types=[
        pltpu.VMEM((_GN, _GCH), jnp.int32),
        pltpu.VMEM((2, _GCH, _D), jnp.float32),
        pltpu.SemaphoreType.DMA,
        pltpu.SemaphoreType.DMA,
    ],
)
def _gather(table_hbm, tok_hbm, out_hbm, idx_v, bufs, sem0, sem1):
    wid = lax.axis_index("s") * _NC + lax.axis_index("c")
    pltpu.sync_copy(tok_hbm.at[pl.ds(wid * _GN, _GN)], idx_v)
    sems = (sem0, sem1)
    cps = [None, None]
    cps[0] = pltpu.async_copy(table_hbm.at[idx_v.at[0]], bufs.at[0], sem0)
    for c in range(_GN):
        cur = c % 2
        if c + 1 < _GN:
            cps[1 - cur] = pltpu.async_copy(
                table_hbm.at[idx_v.at[c + 1]], bufs.at[1 - cur], sems[1 - cur])
        cps[cur].wait()
        pltpu.sync_copy(
            bufs.at[cur],
            out_hbm.at[pl.ds(wid * _ROWS_PER_TILE + c * _GCH, _GCH)])


_MMBLK = 2048


def _matmul(emb, w, b2):
    def body(x_ref, w_ref, b_ref, o_ref):
        o_ref[...] = lax.dot_general(
            x_ref[...], w_ref[...], (((1,), (1,)), ((), ())),
            preferred_element_type=jnp.float32) + b_ref[...]

    return pl.pallas_call(
        body,
        grid=(_N // _MMBLK,),
        in_specs=[
            pl.BlockSpec((_MMBLK, _D), lambda i: (i, 0)),
            pl.BlockSpec((_D, _D), lambda i: (0, 0)),
            pl.BlockSpec((1, _D), lambda i: (0, 0)),
        ],
        out_specs=pl.BlockSpec((_MMBLK, _D), lambda i: (i, 0)),
        out_shape=jax.ShapeDtypeStruct((_N, _D), jnp.float32),
    )(emb, w, b2)


def _make_tree_stage(levels, split, csrc_base, leaf_duty=False):
    first = levels[0]
    out_base = _OFFS[levels[-1] - 1]
    out_rows = _OFFS[first] - out_base

    @functools.partial(
        pl.kernel,
        out_type=[
            jax.ShapeDtypeStruct((out_rows, _D), jnp.float32),
            jax.ShapeDtypeStruct((_NW * _BS, _D), jnp.float32),
        ],
        mesh=_mesh(),
        scratch_types=[
            pltpu.VMEM((5, _GCH), jnp.int32),
            pltpu.VMEM((640, _D), jnp.float32),
            pltpu.VMEM_SHARED((_PMAXROWS + 8, _D), jnp.float32),
            pltpu.VMEM((640,), jnp.int32),
            pltpu.VMEM((_BS, _D), jnp.float32),
            pltpu.SemaphoreType.DMA,
        ],
    )
    def stage(h_hbm, csrc_hbm, pp_hbm, bi_hbm, out_hbm, max_hbm,
              idx2d, buf, partial, bid, acc, sem):
        cid = lax.axis_index("c")
        tid = lax.axis_index("s")
        on0 = cid == 0
        zero16 = jnp.zeros((16,), jnp.float32)
        for r0 in range(_BS):
            for q in range(_D // 16):
                acc[r0, pl.ds(q * 16, 16)] = zero16

        def slot_accumulate(nrows):
            def block16(r0, cnt):
                idvec = bid[pl.ds(r0, 16)]
                for j in range(cnt):
                    b_ = idvec[j]
                    for q in range(_D // 16):
                        sl = pl.ds(q * 16, 16)
                        acc[b_, sl] = jnp.maximum(acc[b_, sl], buf[r0 + j, sl])

            if nrows >= 16:
                def rowfn(r16, carry):
                    block16(r16 * 16, 16)
                    return carry
                lax.fori_loop(0, nrows // 16, rowfn, 0)
            if nrows % 16:
                block16((nrows // 16) * 16, nrows % 16)

        for li, l in enumerate(levels):
            s, n = _OFFS[l], _LEVEL_SIZES[l]
            ps, pn = _OFFS[l - 1], _LEVEL_SIZES[l - 1]
            if li == 0:
                src, sbase = csrc_hbm, csrc_base
            else:
                src, sbase = out_hbm, out_base
            if split:
                half = pn // _NC
                passes = [(cid * half, half)]
                np_ = half
            else:
                assert pn <= _PMAXROWS
                passes = [(0, pn)]
                np_ = pn
            A = min(_NS, n // _GCH) if n >= _GCH else 1
            span = n // A
            full, tail = span // _GCH, span % _GCH

            for lo, np_ in passes:
                clamp = np_ < pn
                ap = max(a for a in range(1, _NS + 1)
                         if np_ % a == 0 and (np_ // a) % 8 == 0)
                pspan = np_ // ap

                def prow(ref, base):
                    off = pl.multiple_of(base + tid * pspan, 8)
                    return ref.at[pl.ds(off, pspan)]

                def srow():
                    off = pl.multiple_of(tid * pspan, 8)
                    return partial.at[pl.ds(off, pspan)]

                bslice = buf.at[pl.ds(0, pspan)]
                if split:
                    pon = tid < ap if ap < _NS else None
                    won = tid < A if A < _NS else None
                else:
                    pon = jnp.logical_and(on0, tid < ap) if ap < _NS else on0
                    won = jnp.logical_and(on0, tid < A) if A < _NS else on0

                def _guard(pred, thunk):
                    if pred is None:
                        thunk()
                    else:
                        pl.when(pred)(thunk)

                def p1(b=ps, lo=lo, prow=prow, srow=srow, bslice=bslice):
                    pltpu.sync_copy(prow(h_hbm, b + lo), bslice)
                    pltpu.sync_copy(bslice, srow())

                _guard(pon, p1)
                plsc.subcore_barrier()

                def p2(s=s, src=src, sbase=sbase, lo=lo, np_=np_,
                       clamp=clamp, span=span, full=full, tail=tail):
                    gbase = pl.multiple_of(s + tid * span, 8)
                    lbase = pl.multiple_of(s - sbase + tid * span, 8)
                    ppcps = [pltpu.async_copy(
                        pp_hbm.at[pl.ds(gbase + j * _GCH, _GCH)], idx2d.at[j],
                        sem) for j in range(full)]
                    pltpu.sync_copy(src.at[pl.ds(lbase, span)],
                                    buf.at[pl.ds(0, span)])
                    for cp in ppcps:
                        cp.wait()
                    if clamp:
                        for j in range(full):
                            for q in range(_GCH // 16):
                                v = idx2d[j, pl.ds(q * 16, 16)]
                                ok = jnp.logical_and(v >= lo, v < lo + np_)
                                idx2d[j, pl.ds(q * 16, 16)] = jnp.where(
                                    ok, v - lo, jnp.int32(np_))
                    for j in range(full):
                        pltpu.sync_copy(buf.at[pl.ds(j * _GCH, _GCH)],
                                        partial.at[idx2d.at[j]], add=True)
                    if tail:
                        def scoped(idxs):
                            pltpu.sync_copy(
                                pp_hbm.at[pl.ds(gbase + full * _GCH, tail)],
                                idxs)
                            pltpu.sync_copy(buf.at[pl.ds(full * _GCH, tail)],
                                            partial.at[idxs], add=True)
                        pl.run_scoped(scoped, pltpu.VMEM((tail,), jnp.int32))

                _guard(won, p2)
                plsc.subcore_barrier()

                def p3(b=ps - out_base, g=ps, lo=lo, pspan=pspan, prow=prow,
                       srow=srow, bslice=bslice):
                    pltpu.sync_copy(srow(), bslice)
                    pltpu.sync_copy(bslice, prow(out_hbm, b + lo))
                    goff = pl.multiple_of(g + lo + tid * pspan, 8)
                    pltpu.sync_copy(bi_hbm.at[pl.ds(goff, pspan)],
                                    bid.at[pl.ds(0, pspan)])
                    slot_accumulate(pspan)

                _guard(pon, p3)
                plsc.subcore_barrier()

        if leaf_duty:
            def leaves():
                lspan = _LEVEL_SIZES[-1] // _NS
                base = pl.multiple_of(_OFFS[-2] + tid * lspan, 8)
                pltpu.sync_copy(h_hbm.at[pl.ds(base, lspan)], buf)
                pltpu.sync_copy(bi_hbm.at[pl.ds(base, lspan)], bid)
                slot_accumulate(lspan)

            pl.when(jnp.logical_not(on0))(leaves)

        wid = cid * _NS + tid
        def pub():
            moff = pl.multiple_of(wid * _BS, 8)
            pltpu.sync_copy(acc, max_hbm.at[pl.ds(moff, _BS)])
        pub()

    return stage


_T1 = _make_tree_stage([7], split=True, csrc_base=0)
_T2 = _make_tree_stage([6], split=True, csrc_base=_OFFS[6])
_T3 = _make_tree_stage([5, 4, 3, 2, 1], split=False,
                       csrc_base=_OFFS[5], leaf_duty=True)


def _merge(m1, m2, m3):
    def body(a_ref, b_ref, c_ref, o_ref):
        out = jnp.zeros((_BS, _D), jnp.float32)
        for ref in (a_ref, b_ref, c_ref):
            x_ = ref[...]
            for k in range(_NW):
                out = jnp.maximum(out, x_[k * _BS:(k + 1) * _BS, :])
        o_ref[...] = out

    spec = pl.BlockSpec((_NW * _BS, _D), lambda i: (0, 0))
    return pl.pallas_call(
        body,
        grid=(1,),
        in_specs=[spec, spec, spec],
        out_specs=pl.BlockSpec((_BS, _D), lambda i: (0, 0)),
        out_shape=jax.ShapeDtypeStruct((_BS, _D), jnp.float32),
    )(m1, m2, m3)


def kernel(tokens, level_offsets, parent_pos, batch_index, bs, table, W, b):
    del level_offsets, bs
    tok2 = tokens.astype(jnp.int32).reshape(_N // _GCH, _GCH)
    pp32 = parent_pos.astype(jnp.int32)
    bi32 = batch_index.astype(jnp.int32)
    emb = _gather(table, tok2)
    h = _matmul(emb, W, b.reshape(1, _D))
    lvl6, mx1 = _T1(h, h, pp32, bi32)
    lvl5, mx2 = _T2(h, lvl6, pp32, bi32)
    _rest, mx3 = _T3(h, lvl5, pp32, bi32)
    return _merge(mx1, mx2, mx3)


# speedup vs baseline: 1.5085x; 1.0057x over previous
import functools

import jax
import jax.numpy as jnp
from jax import lax
from jax.experimental import pallas as pl
from jax.experimental.pallas import tpu as pltpu
from jax.experimental.pallas import tpu_sc as plsc

_LEVEL_SIZES = (16, 48, 192, 768, 3072, 8192, 10240, 10240)
_N = sum(_LEVEL_SIZES)
_D = 128
_BS = 16
_NC = 2
_NS = 16
_NW = _NC * _NS

_OFFS = [0]
for _s in _LEVEL_SIZES:
    _OFFS.append(_OFFS[-1] + _s)

_GCH = 128
_ROWS_PER_TILE = _N // _NW
_GN = _ROWS_PER_TILE // _GCH
_MMB = 512
_PMAXROWS = 5120
_NINT = _OFFS[-2]


def _mesh():
    return plsc.VectorSubcoreMesh(core_axis_name="c", subcore_axis_name="s")


@functools.partial(
    pl.kernel,
    out_type=jax.ShapeDtypeStruct((_N, _D), jnp.float32),
    mesh=_mesh(),
    scratch_types=[
        pltpu.VMEM((_GN, _GCH), jnp.int32),
        pltpu.VMEM((2, _GCH, _D), jnp.float32),
        pltpu.SemaphoreType.DMA,
        pltpu.SemaphoreType.DMA,
    ],
)
def _gather(table_hbm, tok_hbm, out_hbm, idx_v, bufs, sem0, sem1):
    wid = lax.axis_index("s") * _NC + lax.axis_index("c")
    pltpu.sync_copy(tok_hbm.at[pl.ds(wid * _GN, _GN)], idx_v)
    sems = (sem0, sem1)
    cps = [None, None]
    cps[0] = pltpu.async_copy(table_hbm.at[idx_v.at[0]], bufs.at[0], sem0)
    for c in range(_GN):
        cur = c % 2
        if c + 1 < _GN:
            cps[1 - cur] = pltpu.async_copy(
                table_hbm.at[idx_v.at[c + 1]], bufs.at[1 - cur], sems[1 - cur])
        cps[cur].wait()
        pltpu.sync_copy(
            bufs.at[cur],
            out_hbm.at[pl.ds(wid * _ROWS_PER_TILE + c * _GCH, _GCH)])


_MMBLK = 2048


def _matmul(emb, w, b2):
    def body(x_ref, w_ref, b_ref, o_ref):
        o_ref[...] = lax.dot_general(
            x_ref[...], w_ref[...], (((1,), (1,)), ((), ())),
            preferred_element_type=jnp.float32) + b_ref[...]

    return pl.pallas_call(
        body,
        grid=(_N // _MMBLK,),
        in_specs=[
            pl.BlockSpec((_MMBLK, _D), lambda i: (i, 0)),
            pl.BlockSpec((_D, _D), lambda i: (0, 0)),
            pl.BlockSpec((1, _D), lambda i: (0, 0)),
        ],
        out_specs=pl.BlockSpec((_MMBLK, _D), lambda i: (i, 0)),
        out_shape=jax.ShapeDtypeStruct((_N, _D), jnp.float32),
    )(emb, w, b2)


def _make_tree_stage(levels, split, csrc_base, leaf_duty=False):
    first = levels[0]
    out_base = _OFFS[levels[-1] - 1]
    out_rows = _OFFS[first] - out_base

    @functools.partial(
        pl.kernel,
        out_type=[
            jax.ShapeDtypeStruct((out_rows, _D), jnp.float32),
            jax.ShapeDtypeStruct((_NW * _BS, _D), jnp.float32),
        ],
        mesh=_mesh(),
        scratch_types=[
            pltpu.VMEM((5, _GCH), jnp.int32),
            pltpu.VMEM((640, _D), jnp.float32),
            pltpu.VMEM_SHARED((_PMAXROWS + 8, _D), jnp.float32),
            pltpu.VMEM((640,), jnp.int32),
            pltpu.VMEM((_BS, _D), jnp.float32),
            pltpu.SemaphoreType.DMA,
        ],
    )
    def stage(h_hbm, csrc_hbm, pp_hbm, bi_hbm, out_hbm, max_hbm,
              idx2d, buf, partial, bid, acc, sem):
        cid = lax.axis_index("c")
        tid = lax.axis_index("s")
        on0 = cid == 0
        zero16 = jnp.zeros((16,), jnp.float32)
        for r0 in range(_BS):
            for q in range(_D // 16):
                acc[r0, pl.ds(q * 16, 16)] = zero16

        def slot_accumulate(nrows):
            def block16(r0, cnt):
                idvec = bid[pl.ds(r0, 16)]
                bs_ = [idvec[j] for j in range(cnt)]
                for j in range(cnt):
                    for q in range(_D // 16):
                        sl = pl.ds(q * 16, 16)
                        acc[bs_[j], sl] = jnp.maximum(acc[bs_[j], sl],
                                                      buf[r0 + j, sl])

            if nrows >= 16:
                def rowfn(r16, carry):
                    block16(r16 * 16, 16)
                    return carry
                lax.fori_loop(0, nrows // 16, rowfn, 0)
            if nrows % 16:
                block16((nrows // 16) * 16, nrows % 16)

        for li, l in enumerate(levels):
            s, n = _OFFS[l], _LEVEL_SIZES[l]
            ps, pn = _OFFS[l - 1], _LEVEL_SIZES[l - 1]
            if li == 0:
                src, sbase = csrc_hbm, csrc_base
            else:
                src, sbase = out_hbm, out_base
            if split:
                half = pn // _NC
                passes = [(cid * half, half)]
                np_ = half
            else:
                assert pn <= _PMAXROWS
                passes = [(0, pn)]
                np_ = pn
            A = min(_NS, n // _GCH) if n >= _GCH else 1
            span = n // A
            full, tail = span // _GCH, span % _GCH

            for lo, np_ in passes:
                clamp = np_ < pn
                ap = max(a for a in range(1, _NS + 1)
                         if np_ % a == 0 and (np_ // a) % 8 == 0)
                pspan = np_ // ap

                def prow(ref, base):
                    off = pl.multiple_of(base + tid * pspan, 8)
                    return ref.at[pl.ds(off, pspan)]

                def srow():
                    off = pl.multiple_of(tid * pspan, 8)
                    return partial.at[pl.ds(off, pspan)]

                bslice = buf.at[pl.ds(0, pspan)]
                if split:
                    pon = tid < ap if ap < _NS else None
                    won = tid < A if A < _NS else None
                else:
                    pon = jnp.logical_and(on0, tid < ap) if ap < _NS else on0
                    won = jnp.logical_and(on0, tid < A) if A < _NS else on0

                def _guard(pred, thunk):
                    if pred is None:
                        thunk()
                    else:
                        pl.when(pred)(thunk)

                def p1(b=ps, lo=lo, prow=prow, srow=srow):
                    pltpu.sync_copy(prow(h_hbm, b + lo), srow())

                _guard(pon, p1)
                plsc.subcore_barrier()

                def p2(s=s, src=src, sbase=sbase, lo=lo, np_=np_,
                       clamp=clamp, span=span, full=full, tail=tail):
                    gbase = pl.multiple_of(s + tid * span, 8)
                    lbase = pl.multiple_of(s - sbase + tid * span, 8)
                    ppcps = [pltpu.async_copy(
                        pp_hbm.at[pl.ds(gbase + j * _GCH, _GCH)], idx2d.at[j],
                        sem) for j in range(full)]
                    pltpu.sync_copy(src.at[pl.ds(lbase, span)],
                                    buf.at[pl.ds(0, span)])
                    for cp in ppcps:
                        cp.wait()
                    if clamp:
                        for j in range(full):
                            for q in range(_GCH // 16):
                                v = idx2d[j, pl.ds(q * 16, 16)]
                                ok = jnp.logical_and(v >= lo, v < lo + np_)
                                idx2d[j, pl.ds(q * 16, 16)] = jnp.where(
                                    ok, v - lo, jnp.int32(np_))
                    for j in range(full):
                        pltpu.sync_copy(buf.at[pl.ds(j * _GCH, _GCH)],
                                        partial.at[idx2d.at[j]], add=True)
                    if tail:
                        def scoped(idxs):
                            pltpu.sync_copy(
                                pp_hbm.at[pl.ds(gbase + full * _GCH, tail)],
                                idxs)
                            pltpu.sync_copy(buf.at[pl.ds(full * _GCH, tail)],
                                            partial.at[idxs], add=True)
                        pl.run_scoped(scoped, pltpu.VMEM((tail,), jnp.int32))

                _guard(won, p2)
                plsc.subcore_barrier()

                def p3(b=ps - out_base, g=ps, lo=lo, pspan=pspan, prow=prow,
                       srow=srow, bslice=bslice):
                    pltpu.sync_copy(srow(), bslice)
                    pltpu.sync_copy(bslice, prow(out_hbm, b + lo))
                    goff = pl.multiple_of(g + lo + tid * pspan, 8)
                    pltpu.sync_copy(bi_hbm.at[pl.ds(goff, pspan)],
                                    bid.at[pl.ds(0, pspan)])
                    slot_accumulate(pspan)

                _guard(pon, p3)
                plsc.subcore_barrier()

        if leaf_duty:
            def leaves():
                lspan = _LEVEL_SIZES[-1] // _NS
                base = pl.multiple_of(_OFFS[-2] + tid * lspan, 8)
                pltpu.sync_copy(h_hbm.at[pl.ds(base, lspan)], buf)
                pltpu.sync_copy(bi_hbm.at[pl.ds(base, lspan)], bid)
                slot_accumulate(lspan)

            pl.when(jnp.logical_not(on0))(leaves)

        wid = cid * _NS + tid
        def pub():
            moff = pl.multiple_of(wid * _BS, 8)
            pltpu.sync_copy(acc, max_hbm.at[pl.ds(moff, _BS)])
        pub()

    return stage


_T1 = _make_tree_stage([7], split=True, csrc_base=0)
_T2 = _make_tree_stage([6], split=True, csrc_base=_OFFS[6])
_T3 = _make_tree_stage([5, 4, 3, 2, 1], split=False,
                       csrc_base=_OFFS[5], leaf_duty=True)


def _merge(m1, m2, m3):
    def body(a_ref, b_ref, c_ref, o_ref):
        out = jnp.zeros((_BS, _D), jnp.float32)
        for ref in (a_ref, b_ref, c_ref):
            x_ = ref[...]
            for k in range(_NW):
                out = jnp.maximum(out, x_[k * _BS:(k + 1) * _BS, :])
        o_ref[...] = out

    spec = pl.BlockSpec((_NW * _BS, _D), lambda i: (0, 0))
    return pl.pallas_call(
        body,
        grid=(1,),
        in_specs=[spec, spec, spec],
        out_specs=pl.BlockSpec((_BS, _D), lambda i: (0, 0)),
        out_shape=jax.ShapeDtypeStruct((_BS, _D), jnp.float32),
    )(m1, m2, m3)


def kernel(tokens, level_offsets, parent_pos, batch_index, bs, table, W, b):
    del level_offsets, bs
    tok2 = tokens.astype(jnp.int32).reshape(_N // _GCH, _GCH)
    pp32 = parent_pos.astype(jnp.int32)
    bi32 = batch_index.astype(jnp.int32)
    emb = _gather(table, tok2)
    h = _matmul(emb, W, b.reshape(1, _D))
    lvl6, mx1 = _T1(h, h, pp32, bi32)
    lvl5, mx2 = _T2(h, lvl6, pp32, bi32)
    _rest, mx3 = _T3(h, lvl5, pp32, bi32)
    return _merge(mx1, mx2, mx3)


# speedup vs baseline: 1.6114x; 1.0682x over previous
import functools

import jax
import jax.numpy as jnp
from jax import lax
from jax.experimental import pallas as pl
from jax.experimental.pallas import tpu as pltpu
from jax.experimental.pallas import tpu_sc as plsc

_LEVEL_SIZES = (16, 48, 192, 768, 3072, 8192, 10240, 10240)
_N = sum(_LEVEL_SIZES)
_D = 128
_BS = 16
_NC = 2
_NS = 16
_NW = _NC * _NS

_OFFS = [0]
for _s in _LEVEL_SIZES:
    _OFFS.append(_OFFS[-1] + _s)

_GCH = 128
_ROWS_PER_TILE = _N // _NW
_GN = _ROWS_PER_TILE // _GCH
_MMB = 512
_PMAXROWS = 5120
_NINT = _OFFS[-2]


def _mesh():
    return plsc.VectorSubcoreMesh(core_axis_name="c", subcore_axis_name="s")


@functools.partial(
    pl.kernel,
    out_type=jax.ShapeDtypeStruct((_N, _D), jnp.float32),
    mesh=_mesh(),
    scratch_types=[
        pltpu.VMEM((_GN, _GCH), jnp.int32),
        pltpu.VMEM((2, _GCH, _D), jnp.float32),
        pltpu.SemaphoreType.DMA,
        pltpu.SemaphoreType.DMA,
    ],
)
def _gather(table_hbm, tok_hbm, out_hbm, idx_v, bufs, sem0, sem1):
    wid = lax.axis_index("s") * _NC + lax.axis_index("c")
    pltpu.sync_copy(tok_hbm.at[pl.ds(wid * _GN, _GN)], idx_v)
    sems = (sem0, sem1)
    cps = [None, None]
    cps[0] = pltpu.async_copy(table_hbm.at[idx_v.at[0]], bufs.at[0], sem0)
    for c in range(_GN):
        cur = c % 2
        if c + 1 < _GN:
            cps[1 - cur] = pltpu.async_copy(
                table_hbm.at[idx_v.at[c + 1]], bufs.at[1 - cur], sems[1 - cur])
        cps[cur].wait()
        pltpu.sync_copy(
            bufs.at[cur],
            out_hbm.at[pl.ds(wid * _ROWS_PER_TILE + c * _GCH, _GCH)])


_MMBLK = 4096


def _matmul(emb, w, b2):
    def body(x_ref, w_ref, b_ref, o_ref):
        o_ref[...] = lax.dot_general(
            x_ref[...], w_ref[...], (((1,), (1,)), ((), ())),
            preferred_element_type=jnp.float32) + b_ref[...]

    return pl.pallas_call(
        body,
        grid=(_N // _MMBLK,),
        in_specs=[
            pl.BlockSpec((_MMBLK, _D), lambda i: (i, 0)),
            pl.BlockSpec((_D, _D), lambda i: (0, 0)),
            pl.BlockSpec((1, _D), lambda i: (0, 0)),
        ],
        out_specs=pl.BlockSpec((_MMBLK, _D), lambda i: (i, 0)),
        out_shape=jax.ShapeDtypeStruct((_N, _D), jnp.float32),
    )(emb, w, b2)


def _make_tree_stage(levels, split, csrc_base, leaf_duty=False):
    first = levels[0]
    out_base = _OFFS[levels[-1] - 1]
    out_rows = _OFFS[first] - out_base

    @functools.partial(
        pl.kernel,
        out_type=[
            jax.ShapeDtypeStruct((out_rows, _D), jnp.float32),
            jax.ShapeDtypeStruct((_NW * _BS, _D), jnp.float32),
        ],
        mesh=_mesh(),
        scratch_types=[
            pltpu.VMEM((5, _GCH), jnp.int32),
            pltpu.VMEM((640, _D), jnp.float32),
            pltpu.VMEM_SHARED((_PMAXROWS + 8, _D), jnp.float32),
            pltpu.VMEM((640,), jnp.int32),
            pltpu.VMEM((_BS, _D), jnp.float32),
            pltpu.SemaphoreType.DMA,
        ],
    )
    def stage(h_hbm, csrc_hbm, pp_hbm, bi_hbm, out_hbm, max_hbm,
              idx2d, buf, partial, bid, acc, sem):
        cid = lax.axis_index("c")
        tid = lax.axis_index("s")
        on0 = cid == 0
        zero16 = jnp.zeros((16,), jnp.float32)
        for r0 in range(_BS):
            for q in range(_D // 16):
                acc[r0, pl.ds(q * 16, 16)] = zero16

        def slot_accumulate(nrows):
            def block16(r0, cnt):
                idvec = bid[pl.ds(r0, 16)]
                bs_ = [idvec[j] for j in range(cnt)]
                for j in range(cnt):
                    for q in range(_D // 16):
                        sl = pl.ds(q * 16, 16)
                        acc[bs_[j], sl] = jnp.maximum(acc[bs_[j], sl],
                                                      buf[r0 + j, sl])

            if nrows >= 16:
                def rowfn(r16, carry):
                    block16(r16 * 16, 16)
                    return carry
                lax.fori_loop(0, nrows // 16, rowfn, 0)
            if nrows % 16:
                block16((nrows // 16) * 16, nrows % 16)

        for li, l in enumerate(levels):
            s, n = _OFFS[l], _LEVEL_SIZES[l]
            ps, pn = _OFFS[l - 1], _LEVEL_SIZES[l - 1]
            if li == 0:
                src, sbase = csrc_hbm, csrc_base
            else:
                src, sbase = out_hbm, out_base
            if split:
                half = pn // _NC
                passes = [(cid * half, half)]
                np_ = half
            else:
                assert pn <= _PMAXROWS
                passes = [(0, pn)]
                np_ = pn
            A = min(_NS, n // _GCH) if n >= _GCH else 1
            span = n // A
            full, tail = span // _GCH, span % _GCH

            for lo, np_ in passes:
                clamp = np_ < pn
                ap = max(a for a in range(1, _NS + 1)
                         if np_ % a == 0 and (np_ // a) % 8 == 0)
                pspan = np_ // ap

                def prow(ref, base):
                    off = pl.multiple_of(base + tid * pspan, 8)
                    return ref.at[pl.ds(off, pspan)]

                def srow():
                    off = pl.multiple_of(tid * pspan, 8)
                    return partial.at[pl.ds(off, pspan)]

                bslice = buf.at[pl.ds(0, pspan)]
                if split:
                    pon = tid < ap if ap < _NS else None
                    won = tid < A if A < _NS else None
                else:
                    pon = jnp.logical_and(on0, tid < ap) if ap < _NS else on0
                    won = jnp.logical_and(on0, tid < A) if A < _NS else on0

                def _guard(pred, thunk):
                    if pred is None:
                        thunk()
                    else:
                        pl.when(pred)(thunk)

                def p1(b=ps, lo=lo, prow=prow, srow=srow):
                    pltpu.sync_copy(prow(h_hbm, b + lo), srow())

                _guard(pon, p1)
                plsc.subcore_barrier()

                def p2(s=s, src=src, sbase=sbase, lo=lo, np_=np_,
                       clamp=clamp, span=span, full=full, tail=tail):
                    gbase = pl.multiple_of(s + tid * span, 8)
                    lbase = pl.multiple_of(s - sbase + tid * span, 8)
                    ppcps = [pltpu.async_copy(
                        pp_hbm.at[pl.ds(gbase + j * _GCH, _GCH)], idx2d.at[j],
                        sem) for j in range(full)]
                    pltpu.sync_copy(src.at[pl.ds(lbase, span)],
                                    buf.at[pl.ds(0, span)])
                    for cp in ppcps:
                        cp.wait()
                    if clamp:
                        for j in range(full):
                            for q in range(_GCH // 16):
                                v = idx2d[j, pl.ds(q * 16, 16)]
                                ok = jnp.logical_and(v >= lo, v < lo + np_)
                                idx2d[j, pl.ds(q * 16, 16)] = jnp.where(
                                    ok, v - lo, jnp.int32(np_))
                    sccps = [pltpu.async_copy(buf.at[pl.ds(j * _GCH, _GCH)],
                                              partial.at[idx2d.at[j]], sem,
                                              add=True)
                             for j in range(full)]
                    for cp in sccps:
                        cp.wait()
                    if tail:
                        def scoped(idxs):
                            pltpu.sync_copy(
                                pp_hbm.at[pl.ds(gbase + full * _GCH, tail)],
                                idxs)
                            pltpu.sync_copy(buf.at[pl.ds(full * _GCH, tail)],
                                            partial.at[idxs], add=True)
                        pl.run_scoped(scoped, pltpu.VMEM((tail,), jnp.int32))

                _guard(won, p2)
                plsc.subcore_barrier()

                def p3(b=ps - out_base, g=ps, lo=lo, pspan=pspan, prow=prow,
                       srow=srow, bslice=bslice):
                    goff = pl.multiple_of(g + lo + tid * pspan, 8)
                    pltpu.sync_copy(bi_hbm.at[pl.ds(goff, pspan)],
                                    bid.at[pl.ds(0, pspan)])
                    pltpu.sync_copy(srow(), bslice)
                    cp = pltpu.async_copy(bslice, prow(out_hbm, b + lo), sem)
                    slot_accumulate(pspan)
                    cp.wait()

                _guard(pon, p3)
                plsc.subcore_barrier()

        if leaf_duty:
            def leaves():
                lspan = _LEVEL_SIZES[-1] // _NS
                base = pl.multiple_of(_OFFS[-2] + tid * lspan, 8)
                pltpu.sync_copy(h_hbm.at[pl.ds(base, lspan)], buf)
                pltpu.sync_copy(bi_hbm.at[pl.ds(base, lspan)], bid)
                slot_accumulate(lspan)

            pl.when(jnp.logical_not(on0))(leaves)

        wid = cid * _NS + tid
        def pub():
            moff = pl.multiple_of(wid * _BS, 8)
            pltpu.sync_copy(acc, max_hbm.at[pl.ds(moff, _BS)])
        pub()

    return stage


_T1 = _make_tree_stage([7], split=True, csrc_base=0)
_T2 = _make_tree_stage([6], split=True, csrc_base=_OFFS[6])
_T3 = _make_tree_stage([5, 4, 3, 2, 1], split=False,
                       csrc_base=_OFFS[5], leaf_duty=True)


def _merge(m1, m2, m3):
    def body(a_ref, b_ref, c_ref, o_ref):
        out = jnp.zeros((_BS, _D), jnp.float32)
        for ref in (a_ref, b_ref, c_ref):
            x_ = ref[...]
            for k in range(_NW):
                out = jnp.maximum(out, x_[k * _BS:(k + 1) * _BS, :])
        o_ref[...] = out

    spec = pl.BlockSpec((_NW * _BS, _D), lambda i: (0, 0))
    return pl.pallas_call(
        body,
        grid=(1,),
        in_specs=[spec, spec, spec],
        out_specs=pl.BlockSpec((_BS, _D), lambda i: (0, 0)),
        out_shape=jax.ShapeDtypeStruct((_BS, _D), jnp.float32),
    )(m1, m2, m3)


def kernel(tokens, level_offsets, parent_pos, batch_index, bs, table, W, b):
    del level_offsets, bs
    tok2 = tokens.astype(jnp.int32).reshape(_N // _GCH, _GCH)
    pp32 = parent_pos.astype(jnp.int32)
    bi32 = batch_index.astype(jnp.int32)
    emb = _gather(table, tok2)
    h = _matmul(emb, W, b.reshape(1, _D))
    lvl6, mx1 = _T1(h, h, pp32, bi32)
    lvl5, mx2 = _T2(h, lvl6, pp32, bi32)
    _rest, mx3 = _T3(h, lvl5, pp32, bi32)
    return _merge(mx1, mx2, mx3)


# speedup vs baseline: 1.6580x; 1.0289x over previous
import functools

import jax
import jax.numpy as jnp
from jax import lax
from jax.experimental import pallas as pl
from jax.experimental.pallas import tpu as pltpu
from jax.experimental.pallas import tpu_sc as plsc

_LEVEL_SIZES = (16, 48, 192, 768, 3072, 8192, 10240, 10240)
_N = sum(_LEVEL_SIZES)
_D = 128
_BS = 16
_NC = 2
_NS = 16
_NW = _NC * _NS

_OFFS = [0]
for _s in _LEVEL_SIZES:
    _OFFS.append(_OFFS[-1] + _s)

_GCH = 128
_ROWS_PER_TILE = _N // _NW
_GN = _ROWS_PER_TILE // _GCH
_MMB = 512
_PMAXROWS = 5120
_NINT = _OFFS[-2]


def _mesh():
    return plsc.VectorSubcoreMesh(core_axis_name="c", subcore_axis_name="s")


@functools.partial(
    pl.kernel,
    out_type=jax.ShapeDtypeStruct((_N, _D), jnp.float32),
    mesh=_mesh(),
    scratch_types=[
        pltpu.VMEM((_GN, _GCH), jnp.int32),
        pltpu.VMEM((2, _GCH, _D), jnp.float32),
        pltpu.SemaphoreType.DMA,
        pltpu.SemaphoreType.DMA,
    ],
)
def _gather(table_hbm, tok_hbm, out_hbm, idx_v, bufs, sem0, sem1):
    wid = lax.axis_index("s") * _NC + lax.axis_index("c")
    pltpu.sync_copy(tok_hbm.at[pl.ds(wid * _GN, _GN)], idx_v)
    sems = (sem0, sem1)
    cps = [None, None]
    cps[0] = pltpu.async_copy(table_hbm.at[idx_v.at[0]], bufs.at[0], sem0)
    for c in range(_GN):
        cur = c % 2
        if c + 1 < _GN:
            cps[1 - cur] = pltpu.async_copy(
                table_hbm.at[idx_v.at[c + 1]], bufs.at[1 - cur], sems[1 - cur])
        cps[cur].wait()
        pltpu.sync_copy(
            bufs.at[cur],
            out_hbm.at[pl.ds(wid * _ROWS_PER_TILE + c * _GCH, _GCH)])


_MMBLK = 4096


def _matmul(emb, w, b2):
    def body(x_ref, w_ref, b_ref, o_ref):
        o_ref[...] = lax.dot_general(
            x_ref[...], w_ref[...], (((1,), (1,)), ((), ())),
            preferred_element_type=jnp.float32) + b_ref[...]

    return pl.pallas_call(
        body,
        grid=(_N // _MMBLK,),
        in_specs=[
            pl.BlockSpec((_MMBLK, _D), lambda i: (i, 0)),
            pl.BlockSpec((_D, _D), lambda i: (0, 0)),
            pl.BlockSpec((1, _D), lambda i: (0, 0)),
        ],
        out_specs=pl.BlockSpec((_MMBLK, _D), lambda i: (i, 0)),
        out_shape=jax.ShapeDtypeStruct((_N, _D), jnp.float32),
    )(emb, w, b2)


def _make_tree_stage(levels, split, csrc_base, leaf_duty=False):
    first = levels[0]
    out_base = _OFFS[levels[-1] - 1]
    out_rows = _OFFS[first] - out_base

    @functools.partial(
        pl.kernel,
        out_type=[
            jax.ShapeDtypeStruct((out_rows, _D), jnp.float32),
            jax.ShapeDtypeStruct((_NW * _BS, _D), jnp.float32),
        ],
        mesh=_mesh(),
        scratch_types=[
            pltpu.VMEM((5, _GCH), jnp.int32),
            pltpu.VMEM((640, _D), jnp.float32),
            pltpu.VMEM_SHARED((_PMAXROWS + 8, _D), jnp.float32),
            pltpu.VMEM((640,), jnp.int32),
            pltpu.VMEM((_BS, _D), jnp.float32),
            pltpu.SemaphoreType.DMA,
            pltpu.SemaphoreType.DMA,
            pltpu.SemaphoreType.DMA,
        ],
    )
    def stage(h_hbm, csrc_hbm, pp_hbm, bi_hbm, out_hbm, max_hbm,
              idx2d, buf, partial, bid, acc, sem, semp, semc):
        cid = lax.axis_index("c")
        tid = lax.axis_index("s")
        on0 = cid == 0
        zero16 = jnp.zeros((16,), jnp.float32)
        for r0 in range(_BS):
            for q in range(_D // 16):
                acc[r0, pl.ds(q * 16, 16)] = zero16

        def slot_accumulate(nrows):
            def block16(r0, cnt):
                idvec = bid[pl.ds(r0, 16)]
                bs_ = [idvec[j] for j in range(cnt)]
                for j in range(cnt):
                    for q in range(_D // 16):
                        sl = pl.ds(q * 16, 16)
                        acc[bs_[j], sl] = jnp.maximum(acc[bs_[j], sl],
                                                      buf[r0 + j, sl])

            if nrows >= 16:
                def rowfn(r16, carry):
                    block16(r16 * 16, 16)
                    return carry
                lax.fori_loop(0, nrows // 16, rowfn, 0)
            if nrows % 16:
                block16((nrows // 16) * 16, nrows % 16)

        for li, l in enumerate(levels):
            s, n = _OFFS[l], _LEVEL_SIZES[l]
            ps, pn = _OFFS[l - 1], _LEVEL_SIZES[l - 1]
            if li == 0:
                src, sbase = csrc_hbm, csrc_base
            else:
                src, sbase = out_hbm, out_base
            if split:
                half = pn // _NC
                passes = [(cid * half, half)]
                np_ = half
            else:
                assert pn <= _PMAXROWS
                passes = [(0, pn)]
                np_ = pn
            A = min(_NS, n // _GCH) if n >= _GCH else 1
            span = n // A
            full, tail = span // _GCH, span % _GCH

            for lo, np_ in passes:
                clamp = np_ < pn
                ap = max(a for a in range(1, _NS + 1)
                         if np_ % a == 0 and (np_ // a) % 8 == 0)
                pspan = np_ // ap

                def prow(ref, base):
                    off = pl.multiple_of(base + tid * pspan, 8)
                    return ref.at[pl.ds(off, pspan)]

                def srow():
                    off = pl.multiple_of(tid * pspan, 8)
                    return partial.at[pl.ds(off, pspan)]

                bslice = buf.at[pl.ds(0, pspan)]
                if split:
                    pon = tid < ap if ap < _NS else None
                    won = tid < A if A < _NS else None
                else:
                    pon = jnp.logical_and(on0, tid < ap) if ap < _NS else on0
                    won = jnp.logical_and(on0, tid < A) if A < _NS else on0

                def _guard(pred, thunk):
                    if pred is None:
                        thunk()
                    else:
                        pl.when(pred)(thunk)

                gbase = pl.multiple_of(s + tid * span, 8)
                lbase = pl.multiple_of(s - sbase + tid * span, 8)

                def clamp_scatter(lo=lo, np_=np_, clamp=clamp, full=full,
                                  tail=tail, gbase=gbase):
                    if clamp:
                        for j in range(full):
                            for q in range(_GCH // 16):
                                v = idx2d[j, pl.ds(q * 16, 16)]
                                ok = jnp.logical_and(v >= lo, v < lo + np_)
                                idx2d[j, pl.ds(q * 16, 16)] = jnp.where(
                                    ok, v - lo, jnp.int32(np_))
                    sccps = [pltpu.async_copy(buf.at[pl.ds(j * _GCH, _GCH)],
                                              partial.at[idx2d.at[j]], sem,
                                              add=True)
                             for j in range(full)]
                    for cp in sccps:
                        cp.wait()
                    if tail:
                        def scoped(idxs):
                            pltpu.sync_copy(
                                pp_hbm.at[pl.ds(gbase + full * _GCH, tail)],
                                idxs)
                            pltpu.sync_copy(buf.at[pl.ds(full * _GCH, tail)],
                                            partial.at[idxs], add=True)
                        pl.run_scoped(scoped, pltpu.VMEM((tail,), jnp.int32))

                if split:
                    cpp = pltpu.async_copy(prow(h_hbm, ps + lo), srow(),
                                           semp)
                    ppcps = [pltpu.async_copy(
                        pp_hbm.at[pl.ds(gbase + j * _GCH, _GCH)],
                        idx2d.at[j], semc) for j in range(full)]
                    cpc = pltpu.async_copy(src.at[pl.ds(lbase, span)],
                                           buf.at[pl.ds(0, span)], semc)
                    cpp.wait()
                    plsc.subcore_barrier()
                    for cp in ppcps:
                        cp.wait()
                    cpc.wait()
                    clamp_scatter()
                    plsc.subcore_barrier()
                else:
                    def p1(b=ps, lo=lo, prow=prow, srow=srow):
                        pltpu.sync_copy(prow(h_hbm, b + lo), srow())

                    _guard(pon, p1)
                    plsc.subcore_barrier()

                    def p2(full=full, gbase=gbase, lbase=lbase, span=span,
                           cs=clamp_scatter):
                        ppcps = [pltpu.async_copy(
                            pp_hbm.at[pl.ds(gbase + j * _GCH, _GCH)],
                            idx2d.at[j], semc) for j in range(full)]
                        pltpu.sync_copy(src.at[pl.ds(lbase, span)],
                                        buf.at[pl.ds(0, span)])
                        for cp in ppcps:
                            cp.wait()
                        cs()

                    _guard(won, p2)
                    plsc.subcore_barrier()

                def p3(b=ps - out_base, g=ps, lo=lo, pspan=pspan, prow=prow,
                       srow=srow, bslice=bslice):
                    goff = pl.multiple_of(g + lo + tid * pspan, 8)
                    pltpu.sync_copy(bi_hbm.at[pl.ds(goff, pspan)],
                                    bid.at[pl.ds(0, pspan)])
                    pltpu.sync_copy(srow(), bslice)
                    cp = pltpu.async_copy(bslice, prow(out_hbm, b + lo), sem)
                    slot_accumulate(pspan)
                    cp.wait()

                _guard(pon, p3)
                plsc.subcore_barrier()

        if leaf_duty:
            def leaves():
                lspan = _LEVEL_SIZES[-1] // _NS
                base = pl.multiple_of(_OFFS[-2] + tid * lspan, 8)
                pltpu.sync_copy(h_hbm.at[pl.ds(base, lspan)], buf)
                pltpu.sync_copy(bi_hbm.at[pl.ds(base, lspan)], bid)
                slot_accumulate(lspan)

            pl.when(jnp.logical_not(on0))(leaves)

        wid = cid * _NS + tid
        def pub():
            moff = pl.multiple_of(wid * _BS, 8)
            pltpu.sync_copy(acc, max_hbm.at[pl.ds(moff, _BS)])
        pub()

    return stage


_T1 = _make_tree_stage([7], split=True, csrc_base=0)
_T2 = _make_tree_stage([6], split=True, csrc_base=_OFFS[6])
_T3 = _make_tree_stage([5, 4, 3, 2, 1], split=False,
                       csrc_base=_OFFS[5], leaf_duty=True)


def _merge(m1, m2, m3):
    def body(a_ref, b_ref, c_ref, o_ref):
        out = jnp.zeros((_BS, _D), jnp.float32)
        for ref in (a_ref, b_ref, c_ref):
            x_ = ref[...]
            for k in range(_NW):
                out = jnp.maximum(out, x_[k * _BS:(k + 1) * _BS, :])
        o_ref[...] = out

    spec = pl.BlockSpec((_NW * _BS, _D), lambda i: (0, 0))
    return pl.pallas_call(
        body,
        grid=(1,),
        in_specs=[spec, spec, spec],
        out_specs=pl.BlockSpec((_BS, _D), lambda i: (0, 0)),
        out_shape=jax.ShapeDtypeStruct((_BS, _D), jnp.float32),
    )(m1, m2, m3)


def kernel(tokens, level_offsets, parent_pos, batch_index, bs, table, W, b):
    del level_offsets, bs
    tok2 = tokens.astype(jnp.int32).reshape(_N // _GCH, _GCH)
    pp32 = parent_pos.astype(jnp.int32)
    bi32 = batch_index.astype(jnp.int32)
    emb = _gather(table, tok2)
    h = _matmul(emb, W, b.reshape(1, _D))
    lvl6, mx1 = _T1(h, h, pp32, bi32)
    lvl5, mx2 = _T2(h, lvl6, pp32, bi32)
    _rest, mx3 = _T3(h, lvl5, pp32, bi32)
    return _merge(mx1, mx2, mx3)


# speedup vs baseline: 1.6629x; 1.0030x over previous
import functools

import jax
import jax.numpy as jnp
from jax import lax
from jax.experimental import pallas as pl
from jax.experimental.pallas import tpu as pltpu
from jax.experimental.pallas import tpu_sc as plsc

_LEVEL_SIZES = (16, 48, 192, 768, 3072, 8192, 10240, 10240)
_N = sum(_LEVEL_SIZES)
_D = 128
_BS = 16
_NC = 2
_NS = 16
_NW = _NC * _NS

_OFFS = [0]
for _s in _LEVEL_SIZES:
    _OFFS.append(_OFFS[-1] + _s)

_GCH = 128
_ROWS_PER_TILE = _N // _NW
_GN = _ROWS_PER_TILE // _GCH
_MMB = 512
_PMAXROWS = 5120
_NINT = _OFFS[-2]


def _mesh():
    return plsc.VectorSubcoreMesh(core_axis_name="c", subcore_axis_name="s")


@functools.partial(
    pl.kernel,
    out_type=jax.ShapeDtypeStruct((_N, _D), jnp.float32),
    mesh=_mesh(),
    scratch_types=[
        pltpu.VMEM((_GN, _GCH), jnp.int32),
        pltpu.VMEM((2, _GCH, _D), jnp.float32),
        pltpu.SemaphoreType.DMA,
        pltpu.SemaphoreType.DMA,
    ],
)
def _gather(table_hbm, tok_hbm, out_hbm, idx_v, bufs, sem0, sem1):
    wid = lax.axis_index("s") * _NC + lax.axis_index("c")
    pltpu.sync_copy(tok_hbm.at[pl.ds(wid * _GN, _GN)], idx_v)
    sems = (sem0, sem1)
    cps = [None, None]
    cps[0] = pltpu.async_copy(table_hbm.at[idx_v.at[0]], bufs.at[0], sem0)
    for c in range(_GN):
        cur = c % 2
        if c + 1 < _GN:
            cps[1 - cur] = pltpu.async_copy(
                table_hbm.at[idx_v.at[c + 1]], bufs.at[1 - cur], sems[1 - cur])
        cps[cur].wait()
        pltpu.sync_copy(
            bufs.at[cur],
            out_hbm.at[pl.ds(wid * _ROWS_PER_TILE + c * _GCH, _GCH)])


_MMBLK = 4096


def _matmul(emb, w, b2):
    def body(x_ref, w_ref, b_ref, o_ref):
        o_ref[...] = lax.dot_general(
            x_ref[...], w_ref[...], (((1,), (1,)), ((), ())),
            preferred_element_type=jnp.float32) + b_ref[...]

    return pl.pallas_call(
        body,
        grid=(_N // _MMBLK,),
        in_specs=[
            pl.BlockSpec((_MMBLK, _D), lambda i: (i, 0)),
            pl.BlockSpec((_D, _D), lambda i: (0, 0)),
            pl.BlockSpec((1, _D), lambda i: (0, 0)),
        ],
        out_specs=pl.BlockSpec((_MMBLK, _D), lambda i: (i, 0)),
        out_shape=jax.ShapeDtypeStruct((_N, _D), jnp.float32),
    )(emb, w, b2)


def _make_tree_stage(levels, split, csrc_base, leaf_duty=False):
    first = levels[0]
    out_base = _OFFS[levels[-1] - 1]
    out_rows = _OFFS[first] - out_base

    @functools.partial(
        pl.kernel,
        out_type=[
            jax.ShapeDtypeStruct((out_rows, _D), jnp.float32),
            jax.ShapeDtypeStruct((_NW * _BS, _D), jnp.float32),
        ],
        mesh=_mesh(),
        scratch_types=[
            pltpu.VMEM((5, _GCH), jnp.int32),
            pltpu.VMEM((640, _D), jnp.float32),
            pltpu.VMEM_SHARED((_PMAXROWS + 8, _D), jnp.float32),
            pltpu.VMEM((640,), jnp.int32),
            pltpu.VMEM((_BS, _D), jnp.float32),
            pltpu.SemaphoreType.DMA,
            pltpu.SemaphoreType.DMA,
            pltpu.SemaphoreType.DMA,
        ],
    )
    def stage(h_hbm, csrc_hbm, pp_hbm, bi_hbm, out_hbm, max_hbm,
              idx2d, buf, partial, bid, acc, sem, semp, semc):
        cid = lax.axis_index("c")
        tid = lax.axis_index("s")
        on0 = cid == 0
        zero16 = jnp.zeros((16,), jnp.float32)
        for r0 in range(_BS):
            for q in range(_D // 16):
                acc[r0, pl.ds(q * 16, 16)] = zero16

        def slot_accumulate(nrows):
            def block16(r0, cnt):
                idvec = bid[pl.ds(r0, 16)]
                bs_ = [idvec[j] for j in range(cnt)]
                for j in range(cnt):
                    for q in range(_D // 16):
                        sl = pl.ds(q * 16, 16)
                        acc[bs_[j], sl] = jnp.maximum(acc[bs_[j], sl],
                                                      buf[r0 + j, sl])

            if nrows >= 16:
                def rowfn(r16, carry):
                    block16(r16 * 16, 16)
                    return carry
                lax.fori_loop(0, nrows // 16, rowfn, 0)
            if nrows % 16:
                block16((nrows // 16) * 16, nrows % 16)

        def ptile(np_):
            ap = max(a for a in range(1, _NS + 1)
                     if np_ % a == 0 and (np_ // a) % 8 == 0)
            return ap, np_ // ap

        if not split:
            region = {}
            roff = 0
            for l in levels:
                region[l] = roff
                roff += _LEVEL_SIZES[l - 1]
            assert roff <= _PMAXROWS
            for l in levels:
                ps_l, pn_l = _OFFS[l - 1], _LEVEL_SIZES[l - 1]
                ap_l, pspan_l = ptile(pn_l)
                pred = (jnp.logical_and(on0, tid < ap_l) if ap_l < _NS
                        else on0)

                def p1(ps_l=ps_l, pspan_l=pspan_l, r=region[l]):
                    o1 = pl.multiple_of(ps_l + tid * pspan_l, 8)
                    o2 = pl.multiple_of(r + tid * pspan_l, 8)
                    pltpu.sync_copy(h_hbm.at[pl.ds(o1, pspan_l)],
                                    partial.at[pl.ds(o2, pspan_l)])

                pl.when(pred)(p1)
            plsc.subcore_barrier()

        for li, l in enumerate(levels):
            s, n = _OFFS[l], _LEVEL_SIZES[l]
            ps, pn = _OFFS[l - 1], _LEVEL_SIZES[l - 1]
            if li == 0:
                src, sbase = csrc_hbm, csrc_base
            else:
                src, sbase = out_hbm, out_base
            if split:
                half = pn // _NC
                passes = [(cid * half, half)]
                np_ = half
            else:
                assert pn <= _PMAXROWS
                passes = [(0, pn)]
                np_ = pn
            A = min(_NS, n // _GCH) if n >= _GCH else 1
            span = n // A
            full, tail = span // _GCH, span % _GCH

            for lo, np_ in passes:
                clamp = np_ < pn
                ap, pspan = ptile(np_)
                reg = 0 if split else region[l]

                def prow(ref, base):
                    off = pl.multiple_of(base + tid * pspan, 8)
                    return ref.at[pl.ds(off, pspan)]

                def srow():
                    off = pl.multiple_of(reg + tid * pspan, 8)
                    return partial.at[pl.ds(off, pspan)]

                bslice = buf.at[pl.ds(0, pspan)]
                if split:
                    pon = tid < ap if ap < _NS else None
                    won = tid < A if A < _NS else None
                else:
                    pon = jnp.logical_and(on0, tid < ap) if ap < _NS else on0
                    won = jnp.logical_and(on0, tid < A) if A < _NS else on0

                def _guard(pred, thunk):
                    if pred is None:
                        thunk()
                    else:
                        pl.when(pred)(thunk)

                gbase = pl.multiple_of(s + tid * span, 8)
                lbase = pl.multiple_of(s - sbase + tid * span, 8)

                def clamp_scatter(lo=lo, np_=np_, clamp=clamp, full=full,
                                  tail=tail, gbase=gbase, reg=reg):
                    if clamp:
                        for j in range(full):
                            for q in range(_GCH // 16):
                                v = idx2d[j, pl.ds(q * 16, 16)]
                                ok = jnp.logical_and(v >= lo, v < lo + np_)
                                idx2d[j, pl.ds(q * 16, 16)] = jnp.where(
                                    ok, v - lo, jnp.int32(np_))
                    elif reg:
                        for j in range(full):
                            for q in range(_GCH // 16):
                                sl = pl.ds(q * 16, 16)
                                idx2d[j, sl] = idx2d[j, sl] + jnp.int32(reg)
                    sccps = [pltpu.async_copy(buf.at[pl.ds(j * _GCH, _GCH)],
                                              partial.at[idx2d.at[j]], sem,
                                              add=True)
                             for j in range(full)]
                    for cp in sccps:
                        cp.wait()
                    if tail:
                        def scoped(idxs):
                            pltpu.sync_copy(
                                pp_hbm.at[pl.ds(gbase + full * _GCH, tail)],
                                idxs)
                            if reg:
                                for q in range((tail + 15) // 16):
                                    sl = pl.ds(q * 16, 16)
                                    idxs[sl] = idxs[sl] + jnp.int32(reg)
                            pltpu.sync_copy(buf.at[pl.ds(full * _GCH, tail)],
                                            partial.at[idxs], add=True)
                        pl.run_scoped(scoped, pltpu.VMEM((tail,), jnp.int32))

                if split:
                    cpp = pltpu.async_copy(prow(h_hbm, ps + lo), srow(),
                                           semp)
                    ppcps = [pltpu.async_copy(
                        pp_hbm.at[pl.ds(gbase + j * _GCH, _GCH)],
                        idx2d.at[j], semc) for j in range(full)]
                    cpc = pltpu.async_copy(src.at[pl.ds(lbase, span)],
                                           buf.at[pl.ds(0, span)], semc)
                    cpp.wait()
                    plsc.subcore_barrier()
                    for cp in ppcps:
                        cp.wait()
                    cpc.wait()
                    clamp_scatter()
                    plsc.subcore_barrier()
                else:
                    def p2(full=full, gbase=gbase, lbase=lbase, span=span,
                           cs=clamp_scatter):
                        ppcps = [pltpu.async_copy(
                            pp_hbm.at[pl.ds(gbase + j * _GCH, _GCH)],
                            idx2d.at[j], semc) for j in range(full)]
                        pltpu.sync_copy(src.at[pl.ds(lbase, span)],
                                        buf.at[pl.ds(0, span)])
                        for cp in ppcps:
                            cp.wait()
                        cs()

                    _guard(won, p2)
                    plsc.subcore_barrier()

                def p3(b=ps - out_base, g=ps, lo=lo, pspan=pspan, prow=prow,
                       srow=srow, bslice=bslice):
                    goff = pl.multiple_of(g + lo + tid * pspan, 8)
                    pltpu.sync_copy(bi_hbm.at[pl.ds(goff, pspan)],
                                    bid.at[pl.ds(0, pspan)])
                    pltpu.sync_copy(srow(), bslice)
                    cp = pltpu.async_copy(bslice, prow(out_hbm, b + lo), sem)
                    slot_accumulate(pspan)
                    cp.wait()

                _guard(pon, p3)
                plsc.subcore_barrier()

        if leaf_duty:
            def leaves():
                lspan = _LEVEL_SIZES[-1] // _NS
                base = pl.multiple_of(_OFFS[-2] + tid * lspan, 8)
                pltpu.sync_copy(h_hbm.at[pl.ds(base, lspan)], buf)
                pltpu.sync_copy(bi_hbm.at[pl.ds(base, lspan)], bid)
                slot_accumulate(lspan)

            pl.when(jnp.logical_not(on0))(leaves)

        wid = cid * _NS + tid
        def pub():
            moff = pl.multiple_of(wid * _BS, 8)
            pltpu.sync_copy(acc, max_hbm.at[pl.ds(moff, _BS)])
        pub()

    return stage


_T1 = _make_tree_stage([7], split=True, csrc_base=0)
_T2 = _make_tree_stage([6], split=True, csrc_base=_OFFS[6])
_T3 = _make_tree_stage([5, 4, 3, 2, 1], split=False,
                       csrc_base=_OFFS[5], leaf_duty=True)


def _merge(m1, m2, m3):
    def body(a_ref, b_ref, c_ref, o_ref):
        out = jnp.zeros((_BS, _D), jnp.float32)
        for ref in (a_ref, b_ref, c_ref):
            x_ = ref[...]
            for k in range(_NW):
                out = jnp.maximum(out, x_[k * _BS:(k + 1) * _BS, :])
        o_ref[...] = out

    spec = pl.BlockSpec((_NW * _BS, _D), lambda i: (0, 0))
    return pl.pallas_call(
        body,
        grid=(1,),
        in_specs=[spec, spec, spec],
        out_specs=pl.BlockSpec((_BS, _D), lambda i: (0, 0)),
        out_shape=jax.ShapeDtypeStruct((_BS, _D), jnp.float32),
    )(m1, m2, m3)


def kernel(tokens, level_offsets, parent_pos, batch_index, bs, table, W, b):
    del level_offsets, bs
    tok2 = tokens.astype(jnp.int32).reshape(_N // _GCH, _GCH)
    pp32 = parent_pos.astype(jnp.int32)
    bi32 = batch_index.astype(jnp.int32)
    emb = _gather(table, tok2)
    h = _matmul(emb, W, b.reshape(1, _D))
    lvl6, mx1 = _T1(h, h, pp32, bi32)
    lvl5, mx2 = _T2(h, lvl6, pp32, bi32)
    _rest, mx3 = _T3(h, lvl5, pp32, bi32)
    return _merge(mx1, mx2, mx3)
